# Initial kernel scaffold; baseline (speedup 1.0000x reference)
import jax
import jax.numpy as jnp
from jax.experimental import pallas as pl

B = 8
N_IN = 2048
NC = 1024
NH = 512
NM = NH + N_IN
K_PE = 8
F32 = jnp.float32
HIGH = jax.lax.Precision.HIGHEST


def _dot(a, b, prec=None):
    return jax.lax.dot(a, b, precision=prec, preferred_element_type=F32)


def _row(col, n):
    ii = jax.lax.broadcasted_iota(jnp.int32, (n, n), 0)
    jj = jax.lax.broadcasted_iota(jnp.int32, (n, n), 1)
    zero = jnp.zeros((), dtype=col.dtype)
    diag = jnp.where(ii == jj, jnp.broadcast_to(col, (n, n)), zero)
    return jnp.sum(diag, axis=0, keepdims=True)


def _coord_rows(xb, n):
    return jnp.concatenate([_row(xb[:, c:c + 1], n) for c in range(3)], axis=0)


def _coarse_body(g_ref, w1_ref, b1_ref, w2_ref, b2_ref, o_ref):
    h = jnp.maximum(_dot(g_ref[...], w1_ref[...]) + b1_ref[...], 0.0)
    o_ref[...] = _dot(h, w2_ref[...]) + b2_ref[...]


def _hole_body(p1_ref, x_ref, o_ref):
    p1b = p1_ref[0]
    xb = x_ref[0]
    sq1 = jnp.sum(p1b * p1b, axis=1, keepdims=True)
    sq2 = jnp.sum(xb * xb, axis=0, keepdims=True)
    d = sq1 + sq2 - 2.0 * _dot(p1b, xb, HIGH)
    v = jnp.min(d, axis=1, keepdims=True)
    vrow = _row(v, NC)
    ii = jax.lax.broadcasted_iota(jnp.int32, (NC, NC), 0)
    jj = jax.lax.broadcasted_iota(jnp.int32, (NC, NC), 1)
    vi = jnp.broadcast_to(v, (NC, NC))
    vj = jnp.broadcast_to(vrow, (NC, NC))
    beats = (vj > vi) | ((vj == vi) & (jj < ii))
    rank = jnp.sum(beats.astype(F32), axis=1, keepdims=True)
    rrow = _row(rank, NC)
    rr = jax.lax.broadcasted_iota(F32, (NH, NC), 0)
    sel = (jnp.broadcast_to(rrow, (NH, NC)) == rr).astype(F32)
    o_ref[0] = _dot(sel, p1b, HIGH)


def _fps_body(px_ref, py_ref, pz_ref, ox_ref, oy_ref, oz_ref):
    px, py, pz = px_ref[...], py_ref[...], pz_ref[...]
    jj = jax.lax.broadcasted_iota(jnp.int32, (B, NM), 1)

    def body(i, carry):
        dist, far = carry
        mask = jj == far
        cx = jnp.sum(jnp.where(mask, px, 0.0), axis=1, keepdims=True)
        cy = jnp.sum(jnp.where(mask, py, 0.0), axis=1, keepdims=True)
        cz = jnp.sum(jnp.where(mask, pz, 0.0), axis=1, keepdims=True)
        ox_ref[:, pl.ds(i, 1)] = cx
        oy_ref[:, pl.ds(i, 1)] = cy
        oz_ref[:, pl.ds(i, 1)] = cz
        dx, dy, dz = px - cx, py - cy, pz - cz
        d = dx * dx + dy * dy + dz * dz
        dist = jnp.minimum(dist, d)
        m = jnp.max(dist, axis=1, keepdims=True)
        far = jnp.min(jnp.where(dist == m, jj, NM), axis=1, keepdims=True)
        return dist, far

    dist0 = jnp.full((B, NM), 1e10, dtype=F32)
    far0 = jnp.zeros((B, 1), dtype=jnp.int32)
    jax.lax.fori_loop(0, NC, body, (dist0, far0))


def _knn_body(xx_ref, m1w1_ref, m1b1_ref, m1w2_ref, m1b2_ref,
              m2w1_ref, m2b1_ref, knn_ref, f1_ref, f2_ref):
    xb = xx_ref[0]
    sq = jnp.sum(xb * xb, axis=1, keepdims=True)
    sq_row = _row(sq, NC)
    xbt = _coord_rows(xb, NC)
    dd = sq + sq_row - 2.0 * _dot(xb, xbt, HIGH)
    jj = jax.lax.broadcasted_iota(jnp.int32, (NC, NC), 1)
    nb = []
    for k in range(K_PE):
        m = jnp.min(dd, axis=1, keepdims=True)
        ji = jnp.min(jnp.where(dd == m, jj, NC), axis=1, keepdims=True)
        onehot = jj == ji
        if k >= 1:
            knn_ref[0, :, k - 1:k] = ji
        nb.append(_dot(onehot.astype(F32), xb, HIGH))
        dd = jnp.where(onehot, jnp.float32(1e30), dd)
    mean = nb[0]
    for k in range(1, K_PE):
        mean = mean + nb[k]
    mean = mean * (1.0 / K_PE)
    cen = [p - mean for p in nb]
    f2acc = jnp.broadcast_to(m2b1_ref[...], (NC, 32))
    for a in range(3):
        for b in range(3):
            cov_ab = cen[0][:, a:a + 1] * cen[0][:, b:b + 1]
            for k in range(1, K_PE):
                cov_ab = cov_ab + cen[k][:, a:a + 1] * cen[k][:, b:b + 1]
            cov_ab = cov_ab * (1.0 / K_PE)
            f2acc = f2acc + cov_ab * m2w1_ref[3 * a + b:3 * a + b + 1, :]
    f2_ref[0] = jnp.maximum(f2acc, 0.0)
    a1 = jnp.maximum(_dot(xb, m1w1_ref[...]) + m1b1_ref[...], 0.0)
    f1_ref[0] = jnp.maximum(_dot(a1, m1w2_ref[...]) + m1b2_ref[...], 0.0)


def _att_body(knn_ref, f1_ref, f2_ref, xx_ref,
              aw1_ref, ab1_ref, aw2_ref, ab2_ref,
              w1a_ref, w1b_ref, b1_ref, w2e_ref, b2e_ref, w2o_ref, b2o_ref,
              o_ref):
    f1 = f1_ref[0]
    jj = jax.lax.broadcasted_iota(jnp.int32, (NC, NC), 1)
    fks, wks = [], []
    for k in range(K_PE - 1):
        ji = knn_ref[0, :, k:k + 1]
        onehot = (jj == ji).astype(F32)
        nbf = _dot(onehot, f1, HIGH)
        fk = nbf - f1
        h = jnp.maximum(_dot(fk, aw1_ref[...]) + ab1_ref[...], 0.0)
        wk = _dot(h, aw2_ref[...]) + ab2_ref[...]
        fks.append(fk)
        wks.append(wk)
    m = wks[0]
    for k in range(1, K_PE - 1):
        m = jnp.maximum(m, wks[k])
    es = [jnp.exp(w - m) for w in wks]
    s = es[0]
    for k in range(1, K_PE - 1):
        s = s + es[k]
    inv = 1.0 / s
    agg = (es[0] * inv) * fks[0]
    for k in range(1, K_PE - 1):
        agg = agg + (es[k] * inv) * fks[k]
    h3 = jnp.maximum(_dot(agg, w1a_ref[...]) + _dot(f2_ref[0], w1b_ref[...])
                     + b1_ref[...], 0.0)
    de = _dot(h3, w2e_ref[...]) + b2e_ref[...]
    do = _dot(h3, w2o_ref[...]) + b2o_ref[...]
    xb = xx_ref[0]
    o_ref[0, :NC, :] = xb + 0.15 * de
    o_ref[0, NC:, :] = xb + 0.15 * do


def _batch_spec(shape):
    return pl.BlockSpec((1,) + shape, lambda b: (b,) + (0,) * len(shape))


def _full_spec(shape):
    nd = len(shape)
    return pl.BlockSpec(shape, lambda b: (0,) * nd)


def kernel(g, x, lin_W1, lin_b1, lin_W2, lin_b2, m1_W1, m1_b1, m1_W2, m1_b2,
           m2_W1, m2_b1, m3_W1, m3_b1, m3_W2, m3_b2, att_W1, att_b1, att_W2,
           att_b2):
    coarse_flat = pl.pallas_call(
        _coarse_body,
        out_shape=jax.ShapeDtypeStruct((B, 3 * NC), F32),
    )(g, lin_W1, lin_b1.reshape(1, -1), lin_W2, lin_b2.reshape(1, -1))
    p1 = coarse_flat.reshape(B, 3, NC).transpose(0, 2, 1)

    hole = pl.pallas_call(
        _hole_body,
        grid=(B,),
        in_specs=[_batch_spec((NC, 3)), _batch_spec((3, N_IN))],
        out_specs=_batch_spec((NH, 3)),
        out_shape=jax.ShapeDtypeStruct((B, NH, 3), F32),
    )(p1, x)

    pts = jnp.concatenate([hole, x.transpose(0, 2, 1)], axis=1)
    px, py, pz = pts[:, :, 0], pts[:, :, 1], pts[:, :, 2]
    ox, oy, oz = pl.pallas_call(
        _fps_body,
        out_shape=[jax.ShapeDtypeStruct((B, NC), F32)] * 3,
    )(px, py, pz)
    xx = jnp.stack([ox, oy, oz], axis=2)

    knn, f1, f2 = pl.pallas_call(
        _knn_body,
        grid=(B,),
        in_specs=[_batch_spec((NC, 3)), _full_spec((3, 32)), _full_spec((1, 32)),
                  _full_spec((32, 64)), _full_spec((1, 64)),
                  _full_spec((9, 32)), _full_spec((1, 32))],
        out_specs=[_batch_spec((NC, K_PE - 1)), _batch_spec((NC, 64)),
                   _batch_spec((NC, 32))],
        out_shape=[jax.ShapeDtypeStruct((B, NC, K_PE - 1), jnp.int32),
                   jax.ShapeDtypeStruct((B, NC, 64), F32),
                   jax.ShapeDtypeStruct((B, NC, 32), F32)],
    )(xx, m1_W1, m1_b1.reshape(1, -1), m1_W2, m1_b2.reshape(1, -1),
      m2_W1, m2_b1.reshape(1, -1))

    w1a, w1b = m3_W1[:64], m3_W1[64:]
    w2e, w2o = m3_W2[:, 0::2], m3_W2[:, 1::2]
    b2e, b2o = m3_b2[0::2].reshape(1, 3), m3_b2[1::2].reshape(1, 3)
    out = pl.pallas_call(
        _att_body,
        grid=(B,),
        in_specs=[_batch_spec((NC, K_PE - 1)), _batch_spec((NC, 64)),
                  _batch_spec((NC, 32)), _batch_spec((NC, 3)),
                  _full_spec((64, 128)), _full_spec((1, 128)),
                  _full_spec((128, 64)), _full_spec((1, 64)),
                  _full_spec((64, 128)), _full_spec((32, 128)),
                  _full_spec((1, 128)),
                  _full_spec((128, 3)), _full_spec((1, 3)),
                  _full_spec((128, 3)), _full_spec((1, 3))],
        out_specs=_batch_spec((2 * NC, 3)),
        out_shape=jax.ShapeDtypeStruct((B, 2 * NC, 3), F32),
    )(knn, f1, f2, xx, att_W1, att_b1.reshape(1, -1), att_W2,
      att_b2.reshape(1, -1), w1a, w1b, m3_b1.reshape(1, -1),
      w2e, b2e, w2o, b2o)

    return (p1, out)


# speedup vs baseline: 10.3223x; 10.3223x over previous
import jax
import jax.numpy as jnp
from jax.experimental import pallas as pl

B = 8
N_IN = 2048
NC = 1024
NH = 512
NM = NH + N_IN
K_PE = 8
F32 = jnp.float32
HIGH = jax.lax.Precision.HIGHEST


def _dot(a, b, prec=None):
    return jax.lax.dot(a, b, precision=prec, preferred_element_type=F32)


def _row(col, n):
    ii = jax.lax.broadcasted_iota(jnp.int32, (n, n), 0)
    jj = jax.lax.broadcasted_iota(jnp.int32, (n, n), 1)
    zero = jnp.zeros((), dtype=col.dtype)
    diag = jnp.where(ii == jj, jnp.broadcast_to(col, (n, n)), zero)
    return jnp.sum(diag, axis=0, keepdims=True)


def _coord_rows(xb, n):
    return jnp.concatenate([_row(xb[:, c:c + 1], n) for c in range(3)], axis=0)


def _coarse_body(g_ref, w1_ref, b1_ref, w2_ref, b2_ref, o_ref):
    h = jnp.maximum(_dot(g_ref[...], w1_ref[...]) + b1_ref[...], 0.0)
    o_ref[...] = _dot(h, w2_ref[...]) + b2_ref[...]


def _hole_body(p1_ref, x_ref, o_ref):
    p1b = p1_ref[0]
    xb = x_ref[0]
    sq1 = jnp.sum(p1b * p1b, axis=1, keepdims=True)
    sq2 = jnp.sum(xb * xb, axis=0, keepdims=True)
    d = (sq1 + sq2) - 2.0 * _dot(p1b, xb)
    v = jnp.min(d, axis=1, keepdims=True)
    vrow = _row(v, NC)
    ii = jax.lax.broadcasted_iota(jnp.int32, (NC, NC), 0)
    jj = jax.lax.broadcasted_iota(jnp.int32, (NC, NC), 1)
    vi = jnp.broadcast_to(v, (NC, NC))
    vj = jnp.broadcast_to(vrow, (NC, NC))
    beats = (vj > vi) | ((vj == vi) & (jj < ii))
    rank = jnp.sum(beats.astype(F32), axis=1, keepdims=True).astype(jnp.int32)
    rrow = _row(rank, NC)
    rr = jax.lax.broadcasted_iota(jnp.int32, (NH, NC), 0)
    sel = (jnp.broadcast_to(rrow, (NH, NC)) == rr).astype(F32)
    o_ref[0] = _dot(sel, p1b, HIGH)


def _fps_body(px_ref, py_ref, pz_ref, ox_ref, oy_ref, oz_ref):
    px, py, pz = px_ref[...], py_ref[...], pz_ref[...]
    jj = jax.lax.broadcasted_iota(jnp.int32, (B, NM), 1)
    oj = jax.lax.broadcasted_iota(jnp.int32, (B, NC), 1)

    def body(i, carry):
        dist, far, sx, sy, sz = carry
        mask = jj == far
        cx = jnp.sum(jnp.where(mask, px, 0.0), axis=1, keepdims=True)
        cy = jnp.sum(jnp.where(mask, py, 0.0), axis=1, keepdims=True)
        cz = jnp.sum(jnp.where(mask, pz, 0.0), axis=1, keepdims=True)
        omask = oj == i
        sx = jnp.where(omask, cx, sx)
        sy = jnp.where(omask, cy, sy)
        sz = jnp.where(omask, cz, sz)
        dx, dy, dz = px - cx, py - cy, pz - cz
        d = dx * dx + dy * dy + dz * dz
        dist = jnp.minimum(dist, d)
        m = jnp.max(dist, axis=1, keepdims=True)
        far = jnp.min(jnp.where(dist == m, jj, NM), axis=1, keepdims=True)
        return dist, far, sx, sy, sz

    dist0 = jnp.full((B, NM), 1e10, dtype=F32)
    far0 = jnp.zeros((B, 1), dtype=jnp.int32)
    z = jnp.zeros((B, NC), dtype=F32)
    _, _, sx, sy, sz = jax.lax.fori_loop(0, NC, body, (dist0, far0, z, z, z))
    ox_ref[...] = sx
    oy_ref[...] = sy
    oz_ref[...] = sz


def _knn_body(xx_ref, m1w1_ref, m1b1_ref, m1w2_ref, m1b2_ref,
              m2w1_ref, m2b1_ref, knn_ref, f1_ref, f2_ref):
    xb = xx_ref[0]
    sq = jnp.sum(xb * xb, axis=1, keepdims=True)
    sq_row = _row(sq, NC)
    xbt = _coord_rows(xb, NC)
    dd = (sq + sq_row) - 2.0 * _dot(xb, xbt)
    jj = jax.lax.broadcasted_iota(jnp.int32, (NC, NC), 1)
    nb = []
    for k in range(K_PE):
        m = jnp.min(dd, axis=1, keepdims=True)
        ji = jnp.min(jnp.where(dd == m, jj, NC), axis=1, keepdims=True)
        onehot = jj == ji
        if k >= 1:
            knn_ref[0, :, k - 1:k] = ji
        nb.append(_dot(onehot.astype(F32), xb, HIGH))
        dd = jnp.where(onehot, jnp.float32(1e30), dd)
    mean = nb[0]
    for k in range(1, K_PE):
        mean = mean + nb[k]
    mean = mean * (1.0 / K_PE)
    cen = [p - mean for p in nb]
    f2acc = jnp.broadcast_to(m2b1_ref[...], (NC, 32))
    for a in range(3):
        for b in range(3):
            cov_ab = cen[0][:, a:a + 1] * cen[0][:, b:b + 1]
            for k in range(1, K_PE):
                cov_ab = cov_ab + cen[k][:, a:a + 1] * cen[k][:, b:b + 1]
            cov_ab = cov_ab * (1.0 / K_PE)
            f2acc = f2acc + cov_ab * m2w1_ref[3 * a + b:3 * a + b + 1, :]
    f2_ref[0] = jnp.maximum(f2acc, 0.0)
    a1 = jnp.maximum(_dot(xb, m1w1_ref[...]) + m1b1_ref[...], 0.0)
    f1_ref[0] = jnp.maximum(_dot(a1, m1w2_ref[...]) + m1b2_ref[...], 0.0)


def _att_body(knn_ref, f1_ref, f2_ref, xx_ref,
              aw1_ref, ab1_ref, aw2_ref, ab2_ref,
              w1a_ref, w1b_ref, b1_ref, w2e_ref, b2e_ref, w2o_ref, b2o_ref,
              o_ref):
    f1 = f1_ref[0]
    jj = jax.lax.broadcasted_iota(jnp.int32, (NC, NC), 1)
    fks, wks = [], []
    for k in range(K_PE - 1):
        ji = knn_ref[0, :, k:k + 1]
        onehot = (jj == ji).astype(F32)
        nbf = _dot(onehot, f1, HIGH)
        fk = nbf - f1
        h = jnp.maximum(_dot(fk, aw1_ref[...]) + ab1_ref[...], 0.0)
        wk = _dot(h, aw2_ref[...]) + ab2_ref[...]
        fks.append(fk)
        wks.append(wk)
    m = wks[0]
    for k in range(1, K_PE - 1):
        m = jnp.maximum(m, wks[k])
    es = [jnp.exp(w - m) for w in wks]
    s = es[0]
    for k in range(1, K_PE - 1):
        s = s + es[k]
    inv = 1.0 / s
    agg = (es[0] * inv) * fks[0]
    for k in range(1, K_PE - 1):
        agg = agg + (es[k] * inv) * fks[k]
    h3 = jnp.maximum(_dot(agg, w1a_ref[...]) + _dot(f2_ref[0], w1b_ref[...])
                     + b1_ref[...], 0.0)
    de = _dot(h3, w2e_ref[...]) + b2e_ref[...]
    do = _dot(h3, w2o_ref[...]) + b2o_ref[...]
    xb = xx_ref[0]
    o_ref[0, :NC, :] = xb + 0.15 * de
    o_ref[0, NC:, :] = xb + 0.15 * do


def _batch_spec(shape):
    return pl.BlockSpec((1,) + shape, lambda b: (b,) + (0,) * len(shape))


def _full_spec(shape):
    nd = len(shape)
    return pl.BlockSpec(shape, lambda b: (0,) * nd)


def kernel(g, x, lin_W1, lin_b1, lin_W2, lin_b2, m1_W1, m1_b1, m1_W2, m1_b2,
           m2_W1, m2_b1, m3_W1, m3_b1, m3_W2, m3_b2, att_W1, att_b1, att_W2,
           att_b2):
    coarse_flat = pl.pallas_call(
        _coarse_body,
        out_shape=jax.ShapeDtypeStruct((B, 3 * NC), F32),
    )(g, lin_W1, lin_b1.reshape(1, -1), lin_W2, lin_b2.reshape(1, -1))
    p1 = coarse_flat.reshape(B, 3, NC).transpose(0, 2, 1)

    hole = pl.pallas_call(
        _hole_body,
        grid=(B,),
        in_specs=[_batch_spec((NC, 3)), _batch_spec((3, N_IN))],
        out_specs=_batch_spec((NH, 3)),
        out_shape=jax.ShapeDtypeStruct((B, NH, 3), F32),
    )(p1, x)

    pts = jnp.concatenate([hole, x.transpose(0, 2, 1)], axis=1)
    px, py, pz = pts[:, :, 0], pts[:, :, 1], pts[:, :, 2]
    ox, oy, oz = pl.pallas_call(
        _fps_body,
        out_shape=[jax.ShapeDtypeStruct((B, NC), F32)] * 3,
    )(px, py, pz)
    xx = jnp.stack([ox, oy, oz], axis=2)

    knn, f1, f2 = pl.pallas_call(
        _knn_body,
        grid=(B,),
        in_specs=[_batch_spec((NC, 3)), _full_spec((3, 32)), _full_spec((1, 32)),
                  _full_spec((32, 64)), _full_spec((1, 64)),
                  _full_spec((9, 32)), _full_spec((1, 32))],
        out_specs=[_batch_spec((NC, K_PE - 1)), _batch_spec((NC, 64)),
                   _batch_spec((NC, 32))],
        out_shape=[jax.ShapeDtypeStruct((B, NC, K_PE - 1), jnp.int32),
                   jax.ShapeDtypeStruct((B, NC, 64), F32),
                   jax.ShapeDtypeStruct((B, NC, 32), F32)],
    )(xx, m1_W1, m1_b1.reshape(1, -1), m1_W2, m1_b2.reshape(1, -1),
      m2_W1, m2_b1.reshape(1, -1))

    w1a, w1b = m3_W1[:64], m3_W1[64:]
    w2e, w2o = m3_W2[:, 0::2], m3_W2[:, 1::2]
    b2e, b2o = m3_b2[0::2].reshape(1, 3), m3_b2[1::2].reshape(1, 3)
    out = pl.pallas_call(
        _att_body,
        grid=(B,),
        in_specs=[_batch_spec((NC, K_PE - 1)), _batch_spec((NC, 64)),
                  _batch_spec((NC, 32)), _batch_spec((NC, 3)),
                  _full_spec((64, 128)), _full_spec((1, 128)),
                  _full_spec((128, 64)), _full_spec((1, 64)),
                  _full_spec((64, 128)), _full_spec((32, 128)),
                  _full_spec((1, 128)),
                  _full_spec((128, 3)), _full_spec((1, 3)),
                  _full_spec((128, 3)), _full_spec((1, 3))],
        out_specs=_batch_spec((2 * NC, 3)),
        out_shape=jax.ShapeDtypeStruct((B, 2 * NC, 3), F32),
    )(knn, f1, f2, xx, att_W1, att_b1.reshape(1, -1), att_W2,
      att_b2.reshape(1, -1), w1a, w1b, m3_b1.reshape(1, -1),
      w2e, b2e, w2o, b2o)

    return (p1, out)


# speedup vs baseline: 19.7610x; 1.9144x over previous
import jax
import jax.numpy as jnp
from jax.experimental import pallas as pl

B = 8
N_IN = 2048
NC = 1024
NH = 512
NM = NH + N_IN
K_PE = 8
F32 = jnp.float32
HIGH = jax.lax.Precision.HIGHEST
H3 = None


def _dot(a, b, prec=None):
    return jax.lax.dot(a, b, precision=prec, preferred_element_type=F32)


def _row(col, n):
    ii = jax.lax.broadcasted_iota(jnp.int32, (n, n), 0)
    jj = jax.lax.broadcasted_iota(jnp.int32, (n, n), 1)
    zero = jnp.zeros((), dtype=col.dtype)
    diag = jnp.where(ii == jj, jnp.broadcast_to(col, (n, n)), zero)
    return jnp.sum(diag, axis=0, keepdims=True)


def _coord_rows(xb, n):
    return jnp.concatenate([_row(xb[:, c:c + 1], n) for c in range(3)], axis=0)


def _coarse_body(g_ref, w1_ref, b1_ref, w2_ref, b2_ref, o_ref):
    h = jnp.maximum(_dot(g_ref[...], w1_ref[...]) + b1_ref[...], 0.0)
    o_ref[...] = _dot(h, w2_ref[...]) + b2_ref[...]


def _hole_body(p1_ref, x_ref, o_ref):
    p1b = p1_ref[0]
    xb = x_ref[0]
    sq1 = jnp.sum(p1b * p1b, axis=1, keepdims=True)
    sq2 = jnp.sum(xb * xb, axis=0, keepdims=True)
    d = (sq1 + sq2) - 2.0 * _dot(p1b, xb)
    v = jnp.min(d, axis=1, keepdims=True)
    vrow = _row(v, NC)
    ii = jax.lax.broadcasted_iota(jnp.int32, (NC, NC), 0)
    jj = jax.lax.broadcasted_iota(jnp.int32, (NC, NC), 1)
    vi = jnp.broadcast_to(v, (NC, NC))
    vj = jnp.broadcast_to(vrow, (NC, NC))
    beats = (vj > vi) | ((vj == vi) & (jj < ii))
    rank = jnp.sum(beats.astype(F32), axis=1, keepdims=True).astype(jnp.int32)
    rrow = _row(rank, NC)
    rr = jax.lax.broadcasted_iota(jnp.int32, (NH, NC), 0)
    sel = (jnp.broadcast_to(rrow, (NH, NC)) == rr).astype(F32)
    o_ref[0] = _dot(sel, p1b, HIGH)


def _fps_body(px_ref, py_ref, pz_ref, ox_ref, oy_ref, oz_ref):
    px, py, pz = px_ref[...], py_ref[...], pz_ref[...]
    jj = jax.lax.broadcasted_iota(jnp.int32, (B, NM), 1)
    oj = jax.lax.broadcasted_iota(jnp.int32, (B, NC), 1)

    def body(i, carry):
        dist, far, sx, sy, sz = carry
        mask = jj == far
        cx = jnp.sum(jnp.where(mask, px, 0.0), axis=1, keepdims=True)
        cy = jnp.sum(jnp.where(mask, py, 0.0), axis=1, keepdims=True)
        cz = jnp.sum(jnp.where(mask, pz, 0.0), axis=1, keepdims=True)
        omask = oj == i
        sx = jnp.where(omask, cx, sx)
        sy = jnp.where(omask, cy, sy)
        sz = jnp.where(omask, cz, sz)
        dx, dy, dz = px - cx, py - cy, pz - cz
        d = dx * dx + dy * dy + dz * dz
        dist = jnp.minimum(dist, d)
        m = jnp.max(dist, axis=1, keepdims=True)
        far = jnp.min(jnp.where(dist == m, jj, NM), axis=1, keepdims=True)
        return dist, far, sx, sy, sz

    dist0 = jnp.full((B, NM), 1e10, dtype=F32)
    far0 = jnp.zeros((B, 1), dtype=jnp.int32)
    z = jnp.zeros((B, NC), dtype=F32)
    _, _, sx, sy, sz = jax.lax.fori_loop(0, NC, body, (dist0, far0, z, z, z))
    ox_ref[...] = sx
    oy_ref[...] = sy
    oz_ref[...] = sz


def _knn_body(xx_ref, m1w1_ref, m1b1_ref, m1w2_ref, m1b2_ref,
              sa_ref, sb_ref, wc_ref, m2b1_ref, knn_ref, f1_ref, f2_ref):
    xb = xx_ref[0]
    sq = jnp.sum(xb * xb, axis=1, keepdims=True)
    sq_row = _row(sq, NC)
    xbt = _coord_rows(xb, NC)
    dd = (sq + sq_row) - 2.0 * _dot(xb, xbt)
    jj = jax.lax.broadcasted_iota(jnp.int32, (NC, NC), 1)
    nb = []
    for k in range(K_PE):
        m = jnp.min(dd, axis=1, keepdims=True)
        ji = jnp.min(jnp.where(dd == m, jj, NC), axis=1, keepdims=True)
        onehot = jj == ji
        if k >= 1:
            knn_ref[0, :, k - 1:k] = ji
        nb.append(_dot(onehot.astype(F32), xb, H3))
        dd = jnp.where(onehot, jnp.float32(1e30), dd)
    mean = nb[0]
    for k in range(1, K_PE):
        mean = mean + nb[k]
    mean = mean * (1.0 / K_PE)
    X = jnp.concatenate([p - mean for p in nb], axis=1)
    A = _dot(X, sa_ref[...], H3)
    Bm = _dot(X, sb_ref[...], H3)
    f2acc = _dot(A * Bm, wc_ref[...]) + m2b1_ref[...]
    f2_ref[0] = jnp.maximum(f2acc, 0.0)
    a1 = jnp.maximum(_dot(xb, m1w1_ref[...]) + m1b1_ref[...], 0.0)
    f1_ref[0] = jnp.maximum(_dot(a1, m1w2_ref[...]) + m1b2_ref[...], 0.0)


def _att_body(knn_ref, f1_ref, f2_ref, xx_ref,
              aw1_ref, ab1_ref, aw2_ref, ab2_ref,
              w1a_ref, w1b_ref, b1_ref, w2e_ref, b2e_ref, w2o_ref, b2o_ref,
              o_ref):
    f1 = f1_ref[0]
    jj = jax.lax.broadcasted_iota(jnp.int32, (NC, NC), 1)
    fks, wks = [], []
    for k in range(K_PE - 1):
        ji = knn_ref[0, :, k:k + 1]
        onehot = (jj == ji).astype(F32)
        nbf = _dot(onehot, f1, H3)
        fk = nbf - f1
        h = jnp.maximum(_dot(fk, aw1_ref[...]) + ab1_ref[...], 0.0)
        wk = _dot(h, aw2_ref[...]) + ab2_ref[...]
        fks.append(fk)
        wks.append(wk)
    m = wks[0]
    for k in range(1, K_PE - 1):
        m = jnp.maximum(m, wks[k])
    es = [jnp.exp(w - m) for w in wks]
    s = es[0]
    for k in range(1, K_PE - 1):
        s = s + es[k]
    inv = 1.0 / s
    agg = (es[0] * inv) * fks[0]
    for k in range(1, K_PE - 1):
        agg = agg + (es[k] * inv) * fks[k]
    h3 = jnp.maximum(_dot(agg, w1a_ref[...]) + _dot(f2_ref[0], w1b_ref[...])
                     + b1_ref[...], 0.0)
    de = _dot(h3, w2e_ref[...]) + b2e_ref[...]
    do = _dot(h3, w2o_ref[...]) + b2o_ref[...]
    xb = xx_ref[0]
    o_ref[0, :NC, :] = xb + 0.15 * de
    o_ref[0, NC:, :] = xb + 0.15 * do


def _batch_spec(shape):
    return pl.BlockSpec((1,) + shape, lambda b: (b,) + (0,) * len(shape))


def _full_spec(shape):
    nd = len(shape)
    return pl.BlockSpec(shape, lambda b: (0,) * nd)


def kernel(g, x, lin_W1, lin_b1, lin_W2, lin_b2, m1_W1, m1_b1, m1_W2, m1_b2,
           m2_W1, m2_b1, m3_W1, m3_b1, m3_W2, m3_b2, att_W1, att_b1, att_W2,
           att_b2):
    coarse_flat = pl.pallas_call(
        _coarse_body,
        out_shape=jax.ShapeDtypeStruct((B, 3 * NC), F32),
    )(g, lin_W1, lin_b1.reshape(1, -1), lin_W2, lin_b2.reshape(1, -1))
    p1 = coarse_flat.reshape(B, 3, NC).transpose(0, 2, 1)

    hole = pl.pallas_call(
        _hole_body,
        grid=(B,),
        in_specs=[_batch_spec((NC, 3)), _batch_spec((3, N_IN))],
        out_specs=_batch_spec((NH, 3)),
        out_shape=jax.ShapeDtypeStruct((B, NH, 3), F32),
    )(p1, x)

    pts = jnp.concatenate([hole, x.transpose(0, 2, 1)], axis=1)
    px, py, pz = pts[:, :, 0], pts[:, :, 1], pts[:, :, 2]
    ox, oy, oz = pl.pallas_call(
        _fps_body,
        out_shape=[jax.ShapeDtypeStruct((B, NC), F32)] * 3,
    )(px, py, pz)
    xx = jnp.stack([ox, oy, oz], axis=2)

    c = jnp.arange(9 * K_PE)
    kk, rem = c // 9, c % 9
    rows = jnp.arange(3 * K_PE)[:, None]
    sa = (rows == 3 * kk + rem // 3).astype(F32)
    sb = (rows == 3 * kk + rem % 3).astype(F32)
    wc = m2_W1[rem] * (1.0 / K_PE)
    knn, f1, f2 = pl.pallas_call(
        _knn_body,
        grid=(B,),
        in_specs=[_batch_spec((NC, 3)), _full_spec((3, 32)), _full_spec((1, 32)),
                  _full_spec((32, 64)), _full_spec((1, 64)),
                  _full_spec((3 * K_PE, 9 * K_PE)), _full_spec((3 * K_PE, 9 * K_PE)),
                  _full_spec((9 * K_PE, 32)), _full_spec((1, 32))],
        out_specs=[_batch_spec((NC, K_PE - 1)), _batch_spec((NC, 64)),
                   _batch_spec((NC, 32))],
        out_shape=[jax.ShapeDtypeStruct((B, NC, K_PE - 1), jnp.int32),
                   jax.ShapeDtypeStruct((B, NC, 64), F32),
                   jax.ShapeDtypeStruct((B, NC, 32), F32)],
    )(xx, m1_W1, m1_b1.reshape(1, -1), m1_W2, m1_b2.reshape(1, -1),
      sa, sb, wc, m2_b1.reshape(1, -1))

    w1a, w1b = m3_W1[:64], m3_W1[64:]
    w2e, w2o = m3_W2[:, 0::2], m3_W2[:, 1::2]
    b2e, b2o = m3_b2[0::2].reshape(1, 3), m3_b2[1::2].reshape(1, 3)
    out = pl.pallas_call(
        _att_body,
        grid=(B,),
        in_specs=[_batch_spec((NC, K_PE - 1)), _batch_spec((NC, 64)),
                  _batch_spec((NC, 32)), _batch_spec((NC, 3)),
                  _full_spec((64, 128)), _full_spec((1, 128)),
                  _full_spec((128, 64)), _full_spec((1, 64)),
                  _full_spec((64, 128)), _full_spec((32, 128)),
                  _full_spec((1, 128)),
                  _full_spec((128, 3)), _full_spec((1, 3)),
                  _full_spec((128, 3)), _full_spec((1, 3))],
        out_specs=_batch_spec((2 * NC, 3)),
        out_shape=jax.ShapeDtypeStruct((B, 2 * NC, 3), F32),
    )(knn, f1, f2, xx, att_W1, att_b1.reshape(1, -1), att_W2,
      att_b2.reshape(1, -1), w1a, w1b, m3_b1.reshape(1, -1),
      w2e, b2e, w2o, b2o)

    return (p1, out)


# speedup vs baseline: 20.3474x; 1.0297x over previous
import jax
import jax.numpy as jnp
from jax.experimental import pallas as pl

B = 8
N_IN = 2048
NC = 1024
NH = 512
NM = NH + N_IN
K_PE = 8
F32 = jnp.float32
HIGH = jax.lax.Precision.HIGHEST
H3 = None


def _dot(a, b, prec=None):
    return jax.lax.dot(a, b, precision=prec, preferred_element_type=F32)


def _row(col, n):
    ii = jax.lax.broadcasted_iota(jnp.int32, (n, n), 0)
    jj = jax.lax.broadcasted_iota(jnp.int32, (n, n), 1)
    zero = jnp.zeros((), dtype=col.dtype)
    diag = jnp.where(ii == jj, jnp.broadcast_to(col, (n, n)), zero)
    return jnp.sum(diag, axis=0, keepdims=True)


def _coord_rows(xb, n):
    return jnp.concatenate([_row(xb[:, c:c + 1], n) for c in range(3)], axis=0)


def _coarse_body(g_ref, w1_ref, b1_ref, w2_ref, b2_ref, o_ref):
    h = jnp.maximum(_dot(g_ref[...], w1_ref[...]) + b1_ref[...], 0.0)
    o_ref[...] = _dot(h, w2_ref[...]) + b2_ref[...]


def _hole_body(p1_ref, x_ref, o_ref):
    p1b = p1_ref[0]
    xb = x_ref[0]
    sq1 = jnp.sum(p1b * p1b, axis=1, keepdims=True)
    sq2 = jnp.sum(xb * xb, axis=0, keepdims=True)
    d = (sq1 + sq2) - 2.0 * _dot(p1b, xb)
    v = jnp.min(d, axis=1, keepdims=True)
    vrow = _row(v, NC)
    ii = jax.lax.broadcasted_iota(jnp.int32, (NC, NC), 0)
    jj = jax.lax.broadcasted_iota(jnp.int32, (NC, NC), 1)
    vi = jnp.broadcast_to(v, (NC, NC))
    vj = jnp.broadcast_to(vrow, (NC, NC))
    beats = (vj > vi) | ((vj == vi) & (jj < ii))
    rank = jnp.sum(beats.astype(F32), axis=1, keepdims=True).astype(jnp.int32)
    rrow = _row(rank, NC)
    rr = jax.lax.broadcasted_iota(jnp.int32, (NH, NC), 0)
    sel = (jnp.broadcast_to(rrow, (NH, NC)) == rr).astype(F32)
    o_ref[0] = _dot(sel, p1b, HIGH)


def _fps_body(px_ref, py_ref, pz_ref, ox_ref, oy_ref, oz_ref):
    px, py, pz = px_ref[...], py_ref[...], pz_ref[...]
    jj = jax.lax.broadcasted_iota(jnp.int32, (B, NM), 1)
    CH = 128
    cj = jax.lax.broadcasted_iota(jnp.int32, (B, CH), 1)

    def body(i, carry):
        dist, far, sx, sy, sz = carry
        mask = jj == far
        cx = jnp.sum(jnp.where(mask, px, 0.0), axis=1, keepdims=True)
        cy = jnp.sum(jnp.where(mask, py, 0.0), axis=1, keepdims=True)
        cz = jnp.sum(jnp.where(mask, pz, 0.0), axis=1, keepdims=True)
        omask = cj == i
        sx = jnp.where(omask, cx, sx)
        sy = jnp.where(omask, cy, sy)
        sz = jnp.where(omask, cz, sz)
        dx, dy, dz = px - cx, py - cy, pz - cz
        d = dx * dx + dy * dy + dz * dz
        dist = jnp.minimum(dist, d)
        m = jnp.max(dist, axis=1, keepdims=True)
        far = jnp.min(jnp.where(dist == m, jj, NM), axis=1, keepdims=True)
        return dist, far, sx, sy, sz

    dist = jnp.full((B, NM), 1e10, dtype=F32)
    far = jnp.zeros((B, 1), dtype=jnp.int32)
    z = jnp.zeros((B, CH), dtype=F32)
    for o in range(NC // CH):
        dist, far, sx, sy, sz = jax.lax.fori_loop(
            0, CH, body, (dist, far, z, z, z))
        ox_ref[:, o * CH:(o + 1) * CH] = sx
        oy_ref[:, o * CH:(o + 1) * CH] = sy
        oz_ref[:, o * CH:(o + 1) * CH] = sz


def _knn_body(xx_ref, m1w1_ref, m1b1_ref, m1w2_ref, m1b2_ref,
              sa_ref, sb_ref, wc_ref, m2b1_ref, fks_ref, f2_ref):
    xb = xx_ref[0]
    a1 = jnp.maximum(_dot(xb, m1w1_ref[...]) + m1b1_ref[...], 0.0)
    f1 = jnp.maximum(_dot(a1, m1w2_ref[...]) + m1b2_ref[...], 0.0)
    xf = jnp.concatenate([xb, f1], axis=1)
    sq = jnp.sum(xb * xb, axis=1, keepdims=True)
    sq_row = _row(sq, NC)
    xbt = _coord_rows(xb, NC)
    dd = (sq + sq_row) - 2.0 * _dot(xb, xbt)
    jj = jax.lax.broadcasted_iota(jnp.int32, (NC, NC), 1)
    nb = []
    for k in range(K_PE):
        m = jnp.min(dd, axis=1, keepdims=True)
        ji = jnp.min(jnp.where(dd == m, jj, NC), axis=1, keepdims=True)
        onehot = jj == ji
        g = _dot(onehot.astype(F32), xf, H3)
        nb.append(g[:, :3])
        if k >= 1:
            fks_ref[0, :, 64 * (k - 1):64 * k] = g[:, 3:] - f1
        dd = jnp.where(onehot, jnp.float32(1e30), dd)
    mean = nb[0]
    for k in range(1, K_PE):
        mean = mean + nb[k]
    mean = mean * (1.0 / K_PE)
    X = jnp.concatenate([p - mean for p in nb], axis=1)
    A = _dot(X, sa_ref[...], H3)
    Bm = _dot(X, sb_ref[...], H3)
    f2acc = _dot(A * Bm, wc_ref[...]) + m2b1_ref[...]
    f2_ref[0] = jnp.maximum(f2acc, 0.0)


def _att_body(fks_ref, f2_ref, xx_ref,
              aw1_ref, ab1_ref, aw2_ref, ab2_ref,
              w1a_ref, w1b_ref, b1_ref, w2e_ref, b2e_ref, w2o_ref, b2o_ref,
              o_ref):
    fks, wks = [], []
    for k in range(K_PE - 1):
        fk = fks_ref[0, :, 64 * k:64 * (k + 1)]
        h = jnp.maximum(_dot(fk, aw1_ref[...]) + ab1_ref[...], 0.0)
        wk = _dot(h, aw2_ref[...]) + ab2_ref[...]
        fks.append(fk)
        wks.append(wk)
    m = wks[0]
    for k in range(1, K_PE - 1):
        m = jnp.maximum(m, wks[k])
    es = [jnp.exp(w - m) for w in wks]
    s = es[0]
    for k in range(1, K_PE - 1):
        s = s + es[k]
    inv = 1.0 / s
    agg = (es[0] * inv) * fks[0]
    for k in range(1, K_PE - 1):
        agg = agg + (es[k] * inv) * fks[k]
    h3 = jnp.maximum(_dot(agg, w1a_ref[...]) + _dot(f2_ref[0], w1b_ref[...])
                     + b1_ref[...], 0.0)
    de = _dot(h3, w2e_ref[...]) + b2e_ref[...]
    do = _dot(h3, w2o_ref[...]) + b2o_ref[...]
    xb = xx_ref[0]
    o_ref[0, :NC, :] = xb + 0.15 * de
    o_ref[0, NC:, :] = xb + 0.15 * do


def _batch_spec(shape):
    return pl.BlockSpec((1,) + shape, lambda b: (b,) + (0,) * len(shape))


def _full_spec(shape):
    nd = len(shape)
    return pl.BlockSpec(shape, lambda b: (0,) * nd)


def kernel(g, x, lin_W1, lin_b1, lin_W2, lin_b2, m1_W1, m1_b1, m1_W2, m1_b2,
           m2_W1, m2_b1, m3_W1, m3_b1, m3_W2, m3_b2, att_W1, att_b1, att_W2,
           att_b2):
    coarse_flat = pl.pallas_call(
        _coarse_body,
        out_shape=jax.ShapeDtypeStruct((B, 3 * NC), F32),
    )(g, lin_W1, lin_b1.reshape(1, -1), lin_W2, lin_b2.reshape(1, -1))
    p1 = coarse_flat.reshape(B, 3, NC).transpose(0, 2, 1)

    hole = pl.pallas_call(
        _hole_body,
        grid=(B,),
        in_specs=[_batch_spec((NC, 3)), _batch_spec((3, N_IN))],
        out_specs=_batch_spec((NH, 3)),
        out_shape=jax.ShapeDtypeStruct((B, NH, 3), F32),
    )(p1, x)

    pts = jnp.concatenate([hole, x.transpose(0, 2, 1)], axis=1)
    px, py, pz = pts[:, :, 0], pts[:, :, 1], pts[:, :, 2]
    ox, oy, oz = pl.pallas_call(
        _fps_body,
        out_shape=[jax.ShapeDtypeStruct((B, NC), F32)] * 3,
    )(px, py, pz)
    xx = jnp.stack([ox, oy, oz], axis=2)

    c = jnp.arange(9 * K_PE)
    kk, rem = c // 9, c % 9
    rows = jnp.arange(3 * K_PE)[:, None]
    sa = (rows == 3 * kk + rem // 3).astype(F32)
    sb = (rows == 3 * kk + rem % 3).astype(F32)
    wc = m2_W1[rem] * (1.0 / K_PE)
    fks, f2 = pl.pallas_call(
        _knn_body,
        grid=(B,),
        in_specs=[_batch_spec((NC, 3)), _full_spec((3, 32)), _full_spec((1, 32)),
                  _full_spec((32, 64)), _full_spec((1, 64)),
                  _full_spec((3 * K_PE, 9 * K_PE)), _full_spec((3 * K_PE, 9 * K_PE)),
                  _full_spec((9 * K_PE, 32)), _full_spec((1, 32))],
        out_specs=[_batch_spec((NC, 64 * (K_PE - 1))), _batch_spec((NC, 32))],
        out_shape=[jax.ShapeDtypeStruct((B, NC, 64 * (K_PE - 1)), F32),
                   jax.ShapeDtypeStruct((B, NC, 32), F32)],
    )(xx, m1_W1, m1_b1.reshape(1, -1), m1_W2, m1_b2.reshape(1, -1),
      sa, sb, wc, m2_b1.reshape(1, -1))

    w1a, w1b = m3_W1[:64], m3_W1[64:]
    w2e, w2o = m3_W2[:, 0::2], m3_W2[:, 1::2]
    b2e, b2o = m3_b2[0::2].reshape(1, 3), m3_b2[1::2].reshape(1, 3)
    out = pl.pallas_call(
        _att_body,
        grid=(B,),
        in_specs=[_batch_spec((NC, 64 * (K_PE - 1))),
                  _batch_spec((NC, 32)), _batch_spec((NC, 3)),
                  _full_spec((64, 128)), _full_spec((1, 128)),
                  _full_spec((128, 64)), _full_spec((1, 64)),
                  _full_spec((64, 128)), _full_spec((32, 128)),
                  _full_spec((1, 128)),
                  _full_spec((128, 3)), _full_spec((1, 3)),
                  _full_spec((128, 3)), _full_spec((1, 3))],
        out_specs=_batch_spec((2 * NC, 3)),
        out_shape=jax.ShapeDtypeStruct((B, 2 * NC, 3), F32),
    )(fks, f2, xx, att_W1, att_b1.reshape(1, -1), att_W2,
      att_b2.reshape(1, -1), w1a, w1b, m3_b1.reshape(1, -1),
      w2e, b2e, w2o, b2o)

    return (p1, out)


# speedup vs baseline: 20.9156x; 1.0279x over previous
import jax
import jax.numpy as jnp
from jax.experimental import pallas as pl

B = 8
N_IN = 2048
NC = 1024
NH = 512
NM = NH + N_IN
K_PE = 8
F32 = jnp.float32
HIGH = jax.lax.Precision.HIGHEST
H3 = None


def _dot(a, b, prec=None):
    return jax.lax.dot(a, b, precision=prec, preferred_element_type=F32)


def _row(col, n):
    ii = jax.lax.broadcasted_iota(jnp.int32, (n, n), 0)
    jj = jax.lax.broadcasted_iota(jnp.int32, (n, n), 1)
    zero = jnp.zeros((), dtype=col.dtype)
    diag = jnp.where(ii == jj, jnp.broadcast_to(col, (n, n)), zero)
    return jnp.sum(diag, axis=0, keepdims=True)


def _coord_rows(xb, n):
    return jnp.concatenate([_row(xb[:, c:c + 1], n) for c in range(3)], axis=0)


def _coarse_body(g_ref, w1_ref, b1_ref, w2_ref, b2_ref, o_ref):
    h = jnp.maximum(_dot(g_ref[...], w1_ref[...]) + b1_ref[...], 0.0)
    o_ref[...] = _dot(h, w2_ref[...]) + b2_ref[...]


def _hole_body(p1_ref, x_ref, o_ref):
    p1b = p1_ref[0]
    xb = x_ref[0]
    sq1 = jnp.sum(p1b * p1b, axis=1, keepdims=True)
    sq2 = jnp.sum(xb * xb, axis=0, keepdims=True)
    d = (sq1 + sq2) - 2.0 * _dot(p1b, xb)
    v = jnp.min(d, axis=1, keepdims=True)
    vrow = _row(v, NC)
    ii = jax.lax.broadcasted_iota(jnp.int32, (NC, NC), 0)
    jj = jax.lax.broadcasted_iota(jnp.int32, (NC, NC), 1)
    vi = jnp.broadcast_to(v, (NC, NC))
    vj = jnp.broadcast_to(vrow, (NC, NC))
    beats = (vj > vi) | ((vj == vi) & (jj < ii))
    rank = jnp.sum(beats.astype(F32), axis=1, keepdims=True).astype(jnp.int32)
    rrow = _row(rank, NC)
    rr = jax.lax.broadcasted_iota(jnp.int32, (NH, NC), 0)
    sel = (jnp.broadcast_to(rrow, (NH, NC)) == rr).astype(F32)
    o_ref[0] = _dot(sel, p1b, HIGH)


def _fps_body(px_ref, py_ref, pz_ref, ox_ref, oy_ref, oz_ref):
    px, py, pz = px_ref[...], py_ref[...], pz_ref[...]
    jj = jax.lax.broadcasted_iota(jnp.int32, (B, NM), 1)
    CH = 128
    cj = jax.lax.broadcasted_iota(jnp.int32, (B, CH), 1)

    def body(i, carry):
        dist, far, sx, sy, sz = carry
        mask = jj == far
        cx = jnp.sum(jnp.where(mask, px, 0.0), axis=1, keepdims=True)
        cy = jnp.sum(jnp.where(mask, py, 0.0), axis=1, keepdims=True)
        cz = jnp.sum(jnp.where(mask, pz, 0.0), axis=1, keepdims=True)
        omask = cj == i
        sx = jnp.where(omask, cx, sx)
        sy = jnp.where(omask, cy, sy)
        sz = jnp.where(omask, cz, sz)
        dx, dy, dz = px - cx, py - cy, pz - cz
        d = dx * dx + dy * dy + dz * dz
        dist = jnp.minimum(dist, d)
        m = jnp.max(dist, axis=1, keepdims=True)
        far = jnp.min(jnp.where(dist == m, jj, NM), axis=1, keepdims=True)
        return dist, far, sx, sy, sz

    dist = jnp.full((B, NM), 1e10, dtype=F32)
    far = jnp.zeros((B, 1), dtype=jnp.int32)
    z = jnp.zeros((B, CH), dtype=F32)
    for o in range(NC // CH):
        dist, far, sx, sy, sz = jax.lax.fori_loop(
            0, CH, body, (dist, far, z, z, z))
        ox_ref[:, o * CH:(o + 1) * CH] = sx
        oy_ref[:, o * CH:(o + 1) * CH] = sy
        oz_ref[:, o * CH:(o + 1) * CH] = sz


def _knn_body(xx_ref, m1w1_ref, m1b1_ref, m1w2_ref, m1b2_ref,
              sa_ref, sb_ref, wc_ref, m2b1_ref,
              aw1_ref, ab1_ref, aw2_ref, ab2_ref,
              w1a_ref, w1b_ref, b1_ref, w2e_ref, b2e_ref, w2o_ref, b2o_ref,
              o_ref):
    xb = xx_ref[0]
    a1 = jnp.maximum(_dot(xb, m1w1_ref[...]) + m1b1_ref[...], 0.0)
    f1 = jnp.maximum(_dot(a1, m1w2_ref[...]) + m1b2_ref[...], 0.0)
    xf = jnp.concatenate([xb, f1], axis=1)
    sq = jnp.sum(xb * xb, axis=1, keepdims=True)
    sq_row = _row(sq, NC)
    xbt = _coord_rows(xb, NC)
    dd = (sq + sq_row) - 2.0 * _dot(xb, xbt)
    jj = jax.lax.broadcasted_iota(jnp.int32, (NC, NC), 1)
    nb, fks, wks = [], [], []
    for k in range(K_PE):
        m = jnp.min(dd, axis=1, keepdims=True)
        ji = jnp.min(jnp.where(dd == m, jj, NC), axis=1, keepdims=True)
        onehot = jj == ji
        g = _dot(onehot.astype(F32), xf, H3)
        nb.append(g[:, :3])
        if k >= 1:
            fk = g[:, 3:] - f1
            h = jnp.maximum(_dot(fk, aw1_ref[...]) + ab1_ref[...], 0.0)
            wks.append(_dot(h, aw2_ref[...]) + ab2_ref[...])
            fks.append(fk)
        dd = jnp.where(onehot, jnp.float32(1e30), dd)
    mean = nb[0]
    for k in range(1, K_PE):
        mean = mean + nb[k]
    mean = mean * (1.0 / K_PE)
    X = jnp.concatenate([p - mean for p in nb], axis=1)
    A = _dot(X, sa_ref[...], H3)
    Bm = _dot(X, sb_ref[...], H3)
    f2acc = _dot(A * Bm, wc_ref[...]) + m2b1_ref[...]
    f2 = jnp.maximum(f2acc, 0.0)
    m = wks[0]
    for k in range(1, K_PE - 1):
        m = jnp.maximum(m, wks[k])
    es = [jnp.exp(w - m) for w in wks]
    s = es[0]
    for k in range(1, K_PE - 1):
        s = s + es[k]
    inv = 1.0 / s
    agg = (es[0] * inv) * fks[0]
    for k in range(1, K_PE - 1):
        agg = agg + (es[k] * inv) * fks[k]
    h3 = jnp.maximum(_dot(agg, w1a_ref[...]) + _dot(f2, w1b_ref[...])
                     + b1_ref[...], 0.0)
    de = _dot(h3, w2e_ref[...]) + b2e_ref[...]
    do = _dot(h3, w2o_ref[...]) + b2o_ref[...]
    o_ref[0, :NC, :] = xb + 0.15 * de
    o_ref[0, NC:, :] = xb + 0.15 * do


def _batch_spec(shape):
    return pl.BlockSpec((1,) + shape, lambda b: (b,) + (0,) * len(shape))


def _full_spec(shape):
    nd = len(shape)
    return pl.BlockSpec(shape, lambda b: (0,) * nd)


def kernel(g, x, lin_W1, lin_b1, lin_W2, lin_b2, m1_W1, m1_b1, m1_W2, m1_b2,
           m2_W1, m2_b1, m3_W1, m3_b1, m3_W2, m3_b2, att_W1, att_b1, att_W2,
           att_b2):
    coarse_flat = pl.pallas_call(
        _coarse_body,
        out_shape=jax.ShapeDtypeStruct((B, 3 * NC), F32),
    )(g, lin_W1, lin_b1.reshape(1, -1), lin_W2, lin_b2.reshape(1, -1))
    p1 = coarse_flat.reshape(B, 3, NC).transpose(0, 2, 1)

    hole = pl.pallas_call(
        _hole_body,
        grid=(B,),
        in_specs=[_batch_spec((NC, 3)), _batch_spec((3, N_IN))],
        out_specs=_batch_spec((NH, 3)),
        out_shape=jax.ShapeDtypeStruct((B, NH, 3), F32),
    )(p1, x)

    pts = jnp.concatenate([hole, x.transpose(0, 2, 1)], axis=1)
    px, py, pz = pts[:, :, 0], pts[:, :, 1], pts[:, :, 2]
    ox, oy, oz = pl.pallas_call(
        _fps_body,
        out_shape=[jax.ShapeDtypeStruct((B, NC), F32)] * 3,
    )(px, py, pz)
    xx = jnp.stack([ox, oy, oz], axis=2)

    c = jnp.arange(9 * K_PE)
    kk, rem = c // 9, c % 9
    rows = jnp.arange(3 * K_PE)[:, None]
    sa = (rows == 3 * kk + rem // 3).astype(F32)
    sb = (rows == 3 * kk + rem % 3).astype(F32)
    wc = m2_W1[rem] * (1.0 / K_PE)
    w1a, w1b = m3_W1[:64], m3_W1[64:]
    w2e, w2o = m3_W2[:, 0::2], m3_W2[:, 1::2]
    b2e, b2o = m3_b2[0::2].reshape(1, 3), m3_b2[1::2].reshape(1, 3)
    out = pl.pallas_call(
        _knn_body,
        grid=(B,),
        in_specs=[_batch_spec((NC, 3)), _full_spec((3, 32)), _full_spec((1, 32)),
                  _full_spec((32, 64)), _full_spec((1, 64)),
                  _full_spec((3 * K_PE, 9 * K_PE)), _full_spec((3 * K_PE, 9 * K_PE)),
                  _full_spec((9 * K_PE, 32)), _full_spec((1, 32)),
                  _full_spec((64, 128)), _full_spec((1, 128)),
                  _full_spec((128, 64)), _full_spec((1, 64)),
                  _full_spec((64, 128)), _full_spec((32, 128)),
                  _full_spec((1, 128)),
                  _full_spec((128, 3)), _full_spec((1, 3)),
                  _full_spec((128, 3)), _full_spec((1, 3))],
        out_specs=_batch_spec((2 * NC, 3)),
        out_shape=jax.ShapeDtypeStruct((B, 2 * NC, 3), F32),
    )(xx, m1_W1, m1_b1.reshape(1, -1), m1_W2, m1_b2.reshape(1, -1),
      sa, sb, wc, m2_b1.reshape(1, -1),
      att_W1, att_b1.reshape(1, -1), att_W2, att_b2.reshape(1, -1),
      w1a, w1b, m3_b1.reshape(1, -1), w2e, b2e, w2o, b2o)

    return (p1, out)


# speedup vs baseline: 21.0408x; 1.0060x over previous
import jax
import jax.numpy as jnp
from jax.experimental import pallas as pl

B = 8
N_IN = 2048
NC = 1024
NH = 512
NM = NH + N_IN
K_PE = 8
F32 = jnp.float32
HIGH = jax.lax.Precision.HIGHEST
H3 = None


def _dot(a, b, prec=None):
    return jax.lax.dot(a, b, precision=prec, preferred_element_type=F32)


def _row(col, n):
    ii = jax.lax.broadcasted_iota(jnp.int32, (n, n), 0)
    jj = jax.lax.broadcasted_iota(jnp.int32, (n, n), 1)
    zero = jnp.zeros((), dtype=col.dtype)
    diag = jnp.where(ii == jj, jnp.broadcast_to(col, (n, n)), zero)
    return jnp.sum(diag, axis=0, keepdims=True)


def _coord_rows(xb, n):
    return jnp.concatenate([_row(xb[:, c:c + 1], n) for c in range(3)], axis=0)


def _coarse_body(g_ref, w1_ref, b1_ref, w2_ref, b2_ref, o_ref):
    h = jnp.maximum(_dot(g_ref[...], w1_ref[...]) + b1_ref[...], 0.0)
    o_ref[...] = _dot(h, w2_ref[...]) + b2_ref[...]


def _hole_body(p1_ref, x_ref, o_ref):
    p1b = p1_ref[0]
    xb = x_ref[0]
    sq1 = jnp.sum(p1b * p1b, axis=1, keepdims=True)
    sq2 = jnp.sum(xb * xb, axis=0, keepdims=True)
    d = (sq1 + sq2) - 2.0 * _dot(p1b, xb)
    v = jnp.min(d, axis=1, keepdims=True)
    vrow = _row(v, NC)
    ii = jax.lax.broadcasted_iota(jnp.int32, (NC, NC), 0)
    jj = jax.lax.broadcasted_iota(jnp.int32, (NC, NC), 1)
    vi = jnp.broadcast_to(v, (NC, NC))
    vj = jnp.broadcast_to(vrow, (NC, NC))
    beats = (vj > vi) | ((vj == vi) & (jj < ii))
    rank = jnp.sum(beats.astype(F32), axis=1, keepdims=True).astype(jnp.int32)
    rrow = _row(rank, NC)
    rr = jax.lax.broadcasted_iota(jnp.int32, (NH, NC), 0)
    sel = (jnp.broadcast_to(rrow, (NH, NC)) == rr).astype(F32)
    o_ref[0] = _dot(sel, p1b, HIGH)


def _fps_body(px_ref, py_ref, pz_ref, ox_ref, oy_ref, oz_ref):
    px, py, pz = px_ref[...], py_ref[...], pz_ref[...]
    jj = jax.lax.broadcasted_iota(jnp.int32, (B, NM), 1)
    CH = 128
    cj = jax.lax.broadcasted_iota(jnp.int32, (B, CH), 1)

    def step(i, carry):
        dist, far, sx, sy, sz = carry
        mask = jj == far
        cx = jnp.sum(jnp.where(mask, px, 0.0), axis=1, keepdims=True)
        cy = jnp.sum(jnp.where(mask, py, 0.0), axis=1, keepdims=True)
        cz = jnp.sum(jnp.where(mask, pz, 0.0), axis=1, keepdims=True)
        omask = cj == i
        sx = jnp.where(omask, cx, sx)
        sy = jnp.where(omask, cy, sy)
        sz = jnp.where(omask, cz, sz)
        dx, dy, dz = px - cx, py - cy, pz - cz
        d = dx * dx + dy * dy + dz * dz
        dist = jnp.minimum(dist, d)
        m = jnp.max(dist, axis=1, keepdims=True)
        far = jnp.min(jnp.where(dist == m, jj, NM), axis=1, keepdims=True)
        return dist, far, sx, sy, sz

    def body(i2, carry):
        return step(2 * i2 + 1, step(2 * i2, carry))

    dist = jnp.full((B, NM), 1e10, dtype=F32)
    far = jnp.zeros((B, 1), dtype=jnp.int32)
    z = jnp.zeros((B, CH), dtype=F32)
    for o in range(NC // CH):
        dist, far, sx, sy, sz = jax.lax.fori_loop(
            0, CH // 2, body, (dist, far, z, z, z))
        ox_ref[:, o * CH:(o + 1) * CH] = sx
        oy_ref[:, o * CH:(o + 1) * CH] = sy
        oz_ref[:, o * CH:(o + 1) * CH] = sz


def _knn_body(xx_ref, m1w1_ref, m1b1_ref, m1w2_ref, m1b2_ref,
              sa_ref, sb_ref, wc_ref, m2b1_ref,
              aw1_ref, ab1_ref, aw2_ref, ab2_ref,
              w1a_ref, w1b_ref, b1_ref, w2e_ref, b2e_ref, w2o_ref, b2o_ref,
              o_ref):
    xb = xx_ref[0]
    a1 = jnp.maximum(_dot(xb, m1w1_ref[...]) + m1b1_ref[...], 0.0)
    f1 = jnp.maximum(_dot(a1, m1w2_ref[...]) + m1b2_ref[...], 0.0)
    xf = jnp.concatenate([xb, f1], axis=1)
    sq = jnp.sum(xb * xb, axis=1, keepdims=True)
    sq_row = _row(sq, NC)
    xbt = _coord_rows(xb, NC)
    dd = (sq + sq_row) - 2.0 * _dot(xb, xbt)
    jj = jax.lax.broadcasted_iota(jnp.int32, (NC, NC), 1)
    nb, fks, wks = [], [], []
    for k in range(K_PE):
        m = jnp.min(dd, axis=1, keepdims=True)
        ji = jnp.min(jnp.where(dd == m, jj, NC), axis=1, keepdims=True)
        onehot = jj == ji
        g = _dot(onehot.astype(F32), xf, H3)
        nb.append(g[:, :3])
        if k >= 1:
            fk = g[:, 3:] - f1
            h = jnp.maximum(_dot(fk, aw1_ref[...]) + ab1_ref[...], 0.0)
            wks.append(_dot(h, aw2_ref[...]) + ab2_ref[...])
            fks.append(fk)
        dd = jnp.where(onehot, jnp.float32(1e30), dd)
    mean = nb[0]
    for k in range(1, K_PE):
        mean = mean + nb[k]
    mean = mean * (1.0 / K_PE)
    X = jnp.concatenate([p - mean for p in nb], axis=1)
    A = _dot(X, sa_ref[...], H3)
    Bm = _dot(X, sb_ref[...], H3)
    f2acc = _dot(A * Bm, wc_ref[...]) + m2b1_ref[...]
    f2 = jnp.maximum(f2acc, 0.0)
    m = wks[0]
    for k in range(1, K_PE - 1):
        m = jnp.maximum(m, wks[k])
    es = [jnp.exp(w - m) for w in wks]
    s = es[0]
    for k in range(1, K_PE - 1):
        s = s + es[k]
    inv = 1.0 / s
    agg = (es[0] * inv) * fks[0]
    for k in range(1, K_PE - 1):
        agg = agg + (es[k] * inv) * fks[k]
    h3 = jnp.maximum(_dot(agg, w1a_ref[...]) + _dot(f2, w1b_ref[...])
                     + b1_ref[...], 0.0)
    de = _dot(h3, w2e_ref[...]) + b2e_ref[...]
    do = _dot(h3, w2o_ref[...]) + b2o_ref[...]
    o_ref[0, :NC, :] = xb + 0.15 * de
    o_ref[0, NC:, :] = xb + 0.15 * do


def _batch_spec(shape):
    return pl.BlockSpec((1,) + shape, lambda b: (b,) + (0,) * len(shape))


def _full_spec(shape):
    nd = len(shape)
    return pl.BlockSpec(shape, lambda b: (0,) * nd)


def kernel(g, x, lin_W1, lin_b1, lin_W2, lin_b2, m1_W1, m1_b1, m1_W2, m1_b2,
           m2_W1, m2_b1, m3_W1, m3_b1, m3_W2, m3_b2, att_W1, att_b1, att_W2,
           att_b2):
    coarse_flat = pl.pallas_call(
        _coarse_body,
        out_shape=jax.ShapeDtypeStruct((B, 3 * NC), F32),
    )(g, lin_W1, lin_b1.reshape(1, -1), lin_W2, lin_b2.reshape(1, -1))
    p1 = coarse_flat.reshape(B, 3, NC).transpose(0, 2, 1)

    hole = pl.pallas_call(
        _hole_body,
        grid=(B,),
        in_specs=[_batch_spec((NC, 3)), _batch_spec((3, N_IN))],
        out_specs=_batch_spec((NH, 3)),
        out_shape=jax.ShapeDtypeStruct((B, NH, 3), F32),
    )(p1, x)

    pts = jnp.concatenate([hole, x.transpose(0, 2, 1)], axis=1)
    px, py, pz = pts[:, :, 0], pts[:, :, 1], pts[:, :, 2]
    ox, oy, oz = pl.pallas_call(
        _fps_body,
        out_shape=[jax.ShapeDtypeStruct((B, NC), F32)] * 3,
    )(px, py, pz)
    xx = jnp.stack([ox, oy, oz], axis=2)

    c = jnp.arange(9 * K_PE)
    kk, rem = c // 9, c % 9
    rows = jnp.arange(3 * K_PE)[:, None]
    sa = (rows == 3 * kk + rem // 3).astype(F32)
    sb = (rows == 3 * kk + rem % 3).astype(F32)
    wc = m2_W1[rem] * (1.0 / K_PE)
    w1a, w1b = m3_W1[:64], m3_W1[64:]
    w2e, w2o = m3_W2[:, 0::2], m3_W2[:, 1::2]
    b2e, b2o = m3_b2[0::2].reshape(1, 3), m3_b2[1::2].reshape(1, 3)
    out = pl.pallas_call(
        _knn_body,
        grid=(B,),
        in_specs=[_batch_spec((NC, 3)), _full_spec((3, 32)), _full_spec((1, 32)),
                  _full_spec((32, 64)), _full_spec((1, 64)),
                  _full_spec((3 * K_PE, 9 * K_PE)), _full_spec((3 * K_PE, 9 * K_PE)),
                  _full_spec((9 * K_PE, 32)), _full_spec((1, 32)),
                  _full_spec((64, 128)), _full_spec((1, 128)),
                  _full_spec((128, 64)), _full_spec((1, 64)),
                  _full_spec((64, 128)), _full_spec((32, 128)),
                  _full_spec((1, 128)),
                  _full_spec((128, 3)), _full_spec((1, 3)),
                  _full_spec((128, 3)), _full_spec((1, 3))],
        out_specs=_batch_spec((2 * NC, 3)),
        out_shape=jax.ShapeDtypeStruct((B, 2 * NC, 3), F32),
    )(xx, m1_W1, m1_b1.reshape(1, -1), m1_W2, m1_b2.reshape(1, -1),
      sa, sb, wc, m2_b1.reshape(1, -1),
      att_W1, att_b1.reshape(1, -1), att_W2, att_b2.reshape(1, -1),
      w1a, w1b, m3_b1.reshape(1, -1), w2e, b2e, w2o, b2o)

    return (p1, out)
